# Initial kernel scaffold; baseline (speedup 1.0000x reference)
import jax
import jax.numpy as jnp
from jax.experimental import pallas as pl


def kernel(user_emb, item_emb, edge_index, W0, b0, W1, b1):
    raise NotImplementedError("write your pallas kernel here")


# speedup vs baseline: 7.3851x; 7.3851x over previous
import functools

import jax
import jax.numpy as jnp
from jax import lax
from jax.experimental import pallas as pl
from jax.experimental.pallas import tpu as pltpu
from jax.experimental.pallas import tpu_sc as plsc

N_NODES = 10000
EMB = 128
C_DIM = 64
N_EDGES = 320000
AW = EMB + 16

NC = 2
NS = 16
NW = NC * NS
E_PER_TILE = N_EDGES // NW
E_BLK = 40
N_BLKS = E_PER_TILE // E_BLK
N_PAD = 10112
ROWS_PER_SUB = N_PAD // NS


def _z_body(x_ref, w_ref, b_ref, o_ref):
    z = jnp.dot(x_ref[...], w_ref[...], preferred_element_type=jnp.float32)
    z = z + b_ref[...]
    z0 = z[:, :C_DIM]
    z1 = z[:, C_DIM:]
    n0 = jnp.sqrt(jnp.sum(z0 * z0, axis=1, keepdims=True)) + 1e-12
    n1 = jnp.sqrt(jnp.sum(z1 * z1, axis=1, keepdims=True)) + 1e-12
    o_ref[...] = jnp.concatenate([z0 / n0, z1 / n1], axis=1)


def _project(all_emb, W, b):
    R = 2000
    return pl.pallas_call(
        _z_body,
        grid=(N_NODES // R,),
        in_specs=[
            pl.BlockSpec((R, EMB), lambda i: (i, 0)),
            pl.BlockSpec((EMB, EMB), lambda i: (0, 0)),
            pl.BlockSpec((1, EMB), lambda i: (0, 0)),
        ],
        out_specs=pl.BlockSpec((R, EMB), lambda i: (i, 0)),
        out_shape=jax.ShapeDtypeStruct((N_NODES, EMB), jnp.float32),
    )(all_emb, W, b)


def _edge_body(z_hbm, row_hbm, col_hbm, p_hbm, wp_hbm,
               acc, ridx, cidx, zr, zc, msgd, dpair, red, sem):
    c = lax.axis_index("c")
    s = lax.axis_index("s")
    zeros16 = jnp.zeros((16,), jnp.float32)

    def zb(i, carry):
        for k in range(EMB // 16):
            msgd[i, pl.ds(16 * k, 16)] = zeros16
        return carry
    lax.fori_loop(0, E_BLK, zb, 0)

    rbase = s * ROWS_PER_SUB
    _full = ROWS_PER_SUB // E_BLK
    _rem = ROWS_PER_SUB % E_BLK

    def zcp(t, carry):
        pltpu.sync_copy(msgd, acc.at[pl.ds(rbase + t * E_BLK, E_BLK)])
        return carry
    lax.fori_loop(0, _full, zcp, 0)
    if _rem:
        pltpu.sync_copy(msgd.at[pl.ds(0, _rem)], acc.at[pl.ds(rbase + _full * E_BLK, _rem)])
    plsc.subcore_barrier()

    base = (c * NS + s) * E_PER_TILE
    lane = lax.iota(jnp.int32, 16)

    def blk(j, carry):
        off = base + j * E_BLK
        pltpu.sync_copy(row_hbm.at[pl.ds(off, E_BLK)], ridx)
        pltpu.sync_copy(col_hbm.at[pl.ds(off, E_BLK)], cidx)
        pltpu.async_copy(z_hbm.at[ridx], zr, sem).wait()
        pltpu.async_copy(z_hbm.at[cidx], zc, sem).wait()

        def edge(e, ecarry):
            zrv = [zr[e, pl.ds(16 * k, 16)] for k in range(8)]
            zcv = [zc[e, pl.ds(16 * k, 16)] for k in range(8)]
            s0 = zrv[0] * zcv[0] + zrv[1] * zcv[1] + zrv[2] * zcv[2] + zrv[3] * zcv[3]
            s1 = zrv[4] * zcv[4] + zrv[5] * zcv[5] + zrv[6] * zcv[6] + zrv[7] * zcv[7]
            for k in (8, 4, 2, 1):
                red[pl.ds(0, 16)] = s0
                red[pl.ds(32, 16)] = s1
                s0 = s0 + red[pl.ds(k, 16)]
                s1 = s1 + red[pl.ds(32 + k, 16)]
            e0 = s0[0]
            e1 = s1[0]
            l0 = jnp.maximum(e0, 0.2 * e0)
            l1 = jnp.maximum(e1, 0.2 * e1)
            w0 = jnp.exp(jnp.full((16,), l0, jnp.float32))
            w1 = jnp.exp(jnp.full((16,), l1, jnp.float32))
            for k in range(4):
                msgd[e, pl.ds(16 * k, 16)] = w0 * zcv[k]
            for k in range(4, 8):
                msgd[e, pl.ds(16 * k, 16)] = w1 * zcv[k]
            dpair[e, :] = jnp.where(lane == 0, w0,
                                    jnp.where(lane == 1, w1, 0.0))
            return ecarry
        lax.fori_loop(0, E_BLK, edge, 0)

        pltpu.sync_copy(msgd, acc.at[ridx], add=True)
        pltpu.sync_copy(dpair, wp_hbm.at[pl.ds(off, E_BLK)])
        return carry
    lax.fori_loop(0, N_BLKS, blk, 0)
    plsc.subcore_barrier()

    def rcp(t, carry):
        o = rbase + t * E_BLK
        pltpu.sync_copy(acc.at[pl.ds(o, E_BLK)], msgd)
        pltpu.sync_copy(msgd, p_hbm.at[c, pl.ds(o, E_BLK)])
        return carry
    lax.fori_loop(0, _full, rcp, 0)
    if _rem:
        o = rbase + _full * E_BLK
        pltpu.sync_copy(acc.at[pl.ds(o, _rem)], msgd.at[pl.ds(0, _rem)])
        pltpu.sync_copy(msgd.at[pl.ds(0, _rem)], p_hbm.at[c, pl.ds(o, _rem)])


def _dpass_body(row_hbm, wp_hbm, d_hbm, accd, ridx, dpair, sem):
    c = lax.axis_index("c")
    s = lax.axis_index("s")
    zeros16 = jnp.zeros((16,), jnp.float32)

    def zb(i, carry):
        dpair[i, :] = zeros16
        return carry
    lax.fori_loop(0, E_BLK, zb, 0)

    rbase = s * ROWS_PER_SUB
    _full = ROWS_PER_SUB // E_BLK
    _rem = ROWS_PER_SUB % E_BLK

    def zcp(t, carry):
        pltpu.sync_copy(dpair, accd.at[pl.ds(rbase + t * E_BLK, E_BLK)])
        return carry
    lax.fori_loop(0, _full, zcp, 0)
    if _rem:
        pltpu.sync_copy(dpair.at[pl.ds(0, _rem)], accd.at[pl.ds(rbase + _full * E_BLK, _rem)])
    plsc.subcore_barrier()

    base = (c * NS + s) * E_PER_TILE

    def blk(j, carry):
        off = base + j * E_BLK
        pltpu.sync_copy(row_hbm.at[pl.ds(off, E_BLK)], ridx)
        pltpu.sync_copy(wp_hbm.at[pl.ds(off, E_BLK)], dpair)
        pltpu.sync_copy(dpair, accd.at[ridx], add=True)
        return carry
    lax.fori_loop(0, N_BLKS, blk, 0)
    plsc.subcore_barrier()

    def rcp(t, carry):
        o = rbase + t * E_BLK
        pltpu.sync_copy(accd.at[pl.ds(o, E_BLK)], dpair)
        pltpu.sync_copy(dpair, d_hbm.at[c, pl.ds(o, E_BLK)])
        return carry
    lax.fori_loop(0, _full, rcp, 0)
    if _rem:
        o = rbase + _full * E_BLK
        pltpu.sync_copy(accd.at[pl.ds(o, _rem)], dpair.at[pl.ds(0, _rem)])
        pltpu.sync_copy(dpair.at[pl.ds(0, _rem)], d_hbm.at[c, pl.ds(o, _rem)])


@functools.partial(jax.jit)
def _edge_pass(Z, row, col):
    mesh = plsc.VectorSubcoreMesh(core_axis_name="c", subcore_axis_name="s",
                                  num_cores=NC, num_subcores=NS)
    f = pl.kernel(
        _edge_body,
        out_type=[
            jax.ShapeDtypeStruct((NC, N_PAD, EMB), jnp.float32),
            jax.ShapeDtypeStruct((N_EDGES, 16), jnp.float32),
        ],
        mesh=mesh,
        scratch_types=[
            pltpu.VMEM_SHARED((N_PAD, EMB), jnp.float32),
            pltpu.VMEM((E_BLK,), jnp.int32),
            pltpu.VMEM((E_BLK,), jnp.int32),
            pltpu.VMEM((E_BLK, EMB), jnp.float32),
            pltpu.VMEM((E_BLK, EMB), jnp.float32),
            pltpu.VMEM((E_BLK, EMB), jnp.float32),
            pltpu.VMEM((E_BLK, 16), jnp.float32),
            pltpu.VMEM((64,), jnp.float32),
            pltpu.SemaphoreType.DMA,
        ],
    )
    P, WP = f(Z, row, col)
    g = pl.kernel(
        _dpass_body,
        out_type=[
            jax.ShapeDtypeStruct((NC, N_PAD, 16), jnp.float32),
        ],
        mesh=mesh,
        scratch_types=[
            pltpu.VMEM_SHARED((N_PAD, 16), jnp.float32),
            pltpu.VMEM((E_BLK,), jnp.int32),
            pltpu.VMEM((E_BLK, 16), jnp.float32),
            pltpu.SemaphoreType.DMA,
        ],
    )
    (D,) = g(row, WP)
    return P, D


def _comb_body(e_ref, n_ref, d_ref, o_ref):
    R = e_ref.shape[0]
    n = n_ref[0] + n_ref[1]
    d = d_ref[0] + d_ref[1]
    d0 = jnp.broadcast_to(d[:, 0:1], (R, C_DIM))
    d1 = jnp.broadcast_to(d[:, 1:2], (R, C_DIM))
    denom = jnp.concatenate([d0, d1], axis=1) + 1e-12
    o_ref[...] = 0.5 * (e_ref[...] + n / denom)


def _combine(all_emb, P, D):
    R = 2000
    return pl.pallas_call(
        _comb_body,
        grid=(N_NODES // R,),
        in_specs=[
            pl.BlockSpec((R, EMB), lambda i: (i, 0)),
            pl.BlockSpec((NC, R, EMB), lambda i: (0, i, 0)),
            pl.BlockSpec((NC, R, 16), lambda i: (0, i, 0)),
        ],
        out_specs=pl.BlockSpec((R, EMB), lambda i: (i, 0)),
        out_shape=jax.ShapeDtypeStruct((N_NODES, EMB), jnp.float32),
    )(all_emb, P, D)


def kernel(user_emb, item_emb, edge_index, W0, b0, W1, b1):
    all_emb = jnp.concatenate([user_emb, item_emb], axis=0)
    W = jnp.concatenate([W0, W1], axis=1)
    b = jnp.concatenate([b0, b1], axis=1)
    Z = _project(all_emb, W, b)
    row = edge_index[0].astype(jnp.int32)
    col = edge_index[1].astype(jnp.int32)
    P, D = _edge_pass(Z, row, col)
    return _combine(all_emb, P, D)
